# Initial kernel scaffold; baseline (speedup 1.0000x reference)
import jax
import jax.numpy as jnp
from jax.experimental import pallas as pl


def kernel(point_cloud, params):
    raise NotImplementedError("write your pallas kernel here")


# speedup vs baseline: 78.9850x; 78.9850x over previous
import functools

import jax
import jax.numpy as jnp
import numpy as np
from jax import lax
from jax.experimental import pallas as pl

BN_SCALE = np.float32(np.sqrt(np.float32(1.0) + np.float32(1e-5)))
_BIG = float(np.float32(3.0e38))


def _fps_body(npoint, xyz_ref, out_ref):
    x = xyz_ref[0]
    y = xyz_ref[1]
    z = xyz_ref[2]
    B, N = x.shape
    lane = lax.broadcasted_iota(jnp.int32, (B, N), 1)
    sel_lane = lax.broadcasted_iota(jnp.int32, (B, npoint), 1)

    def body(i, st):
        dist, far, ax, ay, az = st
        m = lane == far
        cx = jnp.sum(jnp.where(m, x, 0.0), axis=1, keepdims=True)
        cy = jnp.sum(jnp.where(m, y, 0.0), axis=1, keepdims=True)
        cz = jnp.sum(jnp.where(m, z, 0.0), axis=1, keepdims=True)
        keep = (sel_lane == i)
        ax = ax + jnp.where(keep, cx, 0.0)
        ay = ay + jnp.where(keep, cy, 0.0)
        az = az + jnp.where(keep, cz, 0.0)
        dx = x - cx
        dy = y - cy
        dz = z - cz
        d = dx * dx + dy * dy + dz * dz
        dist = jnp.minimum(dist, d)
        mx = jnp.max(dist, axis=1, keepdims=True)
        far = jnp.min(jnp.where(dist == mx, lane, N), axis=1, keepdims=True)
        return dist, far, ax, ay, az

    dist0 = jnp.full((B, N), 1e10, jnp.float32)
    far0 = jnp.zeros((B, 1), jnp.int32)
    acc0 = jnp.zeros((B, npoint), jnp.float32)
    _, _, ax, ay, az = lax.fori_loop(0, npoint, body, (dist0, far0, acc0, acc0, acc0))
    out_ref[0] = ax
    out_ref[1] = ay
    out_ref[2] = az


def _fps(xyz_pl, npoint):
    _, B, N = xyz_pl.shape
    return pl.pallas_call(
        functools.partial(_fps_body, npoint),
        out_shape=jax.ShapeDtypeStruct((3, B, npoint), jnp.float32),
    )(xyz_pl)


def _sa_knn_body(K, q_ref, ptsT_ref, pts_ref, feats_ref,
                 w0x_ref, w0f_ref, b0_ref, w1_ref, b1_ref, out_ref):
    q = q_ref[0]
    ptsT = ptsT_ref[0]
    pts = pts_ref[0]
    feats = feats_ref[0]
    BLK = q.shape[0]
    N = pts.shape[0]

    d = lax.dot_general(q, ptsT, (((1,), (0,)), ((), ())),
                        preferred_element_type=jnp.float32) * -2.0
    d = d + jnp.sum(q * q, axis=1, keepdims=True)
    d = d + jnp.sum(ptsT * ptsT, axis=0, keepdims=True)

    lane = lax.broadcasted_iota(jnp.int32, (BLK, N), 1)
    acc = None
    for _ in range(K):
        mn = jnp.min(d, axis=1, keepdims=True)
        sel = jnp.min(jnp.where(d == mn, lane, N), axis=1, keepdims=True)
        ohb = lane == sel
        d = jnp.where(ohb, _BIG, d)
        oh = ohb.astype(jnp.float32)
        gx = jnp.dot(oh, pts, preferred_element_type=jnp.float32)
        gf = jnp.dot(oh, feats, preferred_element_type=jnp.float32)
        rel = gx - q
        h = jnp.dot(rel, w0x_ref[...], preferred_element_type=jnp.float32)
        h = h + jnp.dot(gf, w0f_ref[...], preferred_element_type=jnp.float32)
        h = jnp.maximum(h + b0_ref[...], 0.0)
        h2 = jnp.dot(h, w1_ref[...], preferred_element_type=jnp.float32) + b1_ref[...]
        acc = h2 if acc is None else jnp.maximum(acc, h2)
    out_ref[0] = acc


def _sa_knn(new_xyz, xyzT, xyz, feats, w0, b0, w1, b1, K=16, blk=128):
    B, S, _ = new_xyz.shape
    blk = min(blk, S)
    N = xyz.shape[1]
    C = feats.shape[2]
    C1, C2 = w0.shape[0], w1.shape[0]
    w0x = jnp.transpose(w0[:, :3])
    w0f = jnp.transpose(w0[:, 3:])
    w1t = jnp.transpose(w1)
    grid = (B, S // blk)
    return pl.pallas_call(
        functools.partial(_sa_knn_body, K),
        grid=grid,
        in_specs=[
            pl.BlockSpec((1, blk, 3), lambda b, s: (b, s, 0)),
            pl.BlockSpec((1, 3, N), lambda b, s: (b, 0, 0)),
            pl.BlockSpec((1, N, 3), lambda b, s: (b, 0, 0)),
            pl.BlockSpec((1, N, C), lambda b, s: (b, 0, 0)),
            pl.BlockSpec((3, C1), lambda b, s: (0, 0)),
            pl.BlockSpec((C, C1), lambda b, s: (0, 0)),
            pl.BlockSpec((1, C1), lambda b, s: (0, 0)),
            pl.BlockSpec((C1, C2), lambda b, s: (0, 0)),
            pl.BlockSpec((1, C2), lambda b, s: (0, 0)),
        ],
        out_specs=pl.BlockSpec((1, blk, C2), lambda b, s: (b, s, 0)),
        out_shape=jax.ShapeDtypeStruct((B, S, C2), jnp.float32),
    )(new_xyz, xyzT, xyz, feats, w0x, w0f, b0.reshape(1, C1), w1t, b1.reshape(1, C2))


def _qkv_body(x_ref, ws_ref, bs_ref, wq_ref, bq_ref, wk_ref, bk_ref,
              wv_ref, bv_ref, q_ref, k_ref, v_ref):
    x = x_ref[0]
    s = jnp.dot(x, ws_ref[...], preferred_element_type=jnp.float32) + bs_ref[...]
    q_ref[0] = jnp.dot(s, wq_ref[...], preferred_element_type=jnp.float32) + bq_ref[...]
    k_ref[0] = jnp.dot(s, wk_ref[...], preferred_element_type=jnp.float32) + bk_ref[...]
    v_ref[0] = jnp.dot(s, wv_ref[...], preferred_element_type=jnp.float32) + bv_ref[...]


def _qkv(x, p, pre, blk=256):
    B, N, Cin = x.shape
    blk = min(blk, N)
    Cm = p[pre + '_ws'].shape[0]
    grid = (B, N // blk)
    wspec = lambda shp: pl.BlockSpec(shp, lambda b, s: (0, 0))
    out = pl.pallas_call(
        _qkv_body,
        grid=grid,
        in_specs=[pl.BlockSpec((1, blk, Cin), lambda b, s: (b, s, 0)),
                  wspec((Cin, Cm)), wspec((1, Cm)), wspec((Cm, Cm)), wspec((1, Cm)),
                  wspec((Cm, Cm)), wspec((1, Cm)), wspec((Cm, Cm)), wspec((1, Cm))],
        out_specs=[pl.BlockSpec((1, blk, Cm), lambda b, s: (b, s, 0))] * 3,
        out_shape=[jax.ShapeDtypeStruct((B, N, Cm), jnp.float32)] * 3,
    )(x, jnp.transpose(p[pre + '_ws']), p[pre + '_bs'].reshape(1, Cm),
      jnp.transpose(p[pre + '_wq']), p[pre + '_bq'].reshape(1, Cm),
      jnp.transpose(p[pre + '_wk']), p[pre + '_bk'].reshape(1, Cm),
      jnp.transpose(p[pre + '_wv']), p[pre + '_bv'].reshape(1, Cm))
    return out


def _attn_body(K, x_ref, pos_ref, posT_ref, posr_ref, q_ref, kall_ref, v_ref,
               pw0_ref, pb0_ref, pw1_ref, pb1_ref,
               aw0_ref, ab0_ref, aw1_ref, ab1_ref, we_ref, be_ref, out_ref):
    xid = x_ref[0]
    posq = pos_ref[0]
    posT = posT_ref[0]
    posr = posr_ref[0]
    q = q_ref[0]
    kall = kall_ref[0]
    v = v_ref[0]
    BLK = posq.shape[0]
    N = posr.shape[0]

    d = lax.dot_general(posq, posT, (((1,), (0,)), ((), ())),
                        preferred_element_type=jnp.float32) * -2.0
    d = d + jnp.sum(posq * posq, axis=1, keepdims=True)
    d = d + jnp.sum(posT * posT, axis=0, keepdims=True)

    lane = lax.broadcasted_iota(jnp.int32, (BLK, N), 1)
    logits = []
    vpes = []
    for _ in range(K):
        mn = jnp.min(d, axis=1, keepdims=True)
        sel = jnp.min(jnp.where(d == mn, lane, N), axis=1, keepdims=True)
        ohb = lane == sel
        d = jnp.where(ohb, _BIG, d)
        oh = ohb.astype(jnp.float32)
        kt = jnp.dot(oh, kall, preferred_element_type=jnp.float32)
        gpos = jnp.dot(oh, posr, preferred_element_type=jnp.float32)
        prel = posq - gpos
        pe = jnp.dot(prel, pw0_ref[...], preferred_element_type=jnp.float32) + pb0_ref[...]
        pe = jnp.maximum(pe / BN_SCALE, 0.0)
        pe = jnp.dot(pe, pw1_ref[...], preferred_element_type=jnp.float32) + pb1_ref[...]
        u = q - kt + pe
        a = jnp.dot(u, aw0_ref[...], preferred_element_type=jnp.float32) + ab0_ref[...]
        a = jnp.maximum(a / BN_SCALE, 0.0)
        a = jnp.dot(a, aw1_ref[...], preferred_element_type=jnp.float32) + ab1_ref[...]
        logits.append(a)
        vpes.append(v + pe)

    m = logits[0]
    for t in range(1, K):
        m = jnp.maximum(m, logits[t])
    num = None
    den = None
    for t in range(K):
        e = jnp.exp(logits[t] - m)
        num = e * vpes[t] if num is None else num + e * vpes[t]
        den = e if den is None else den + e
    agg = num / den
    out_ref[0] = xid + jnp.dot(agg, we_ref[...], preferred_element_type=jnp.float32) + be_ref[...]


def _transformer(x, pos, posT, p, pre, K=16, blk=128):
    B, N, Cin = x.shape
    blk = min(blk, N)
    q, k, v = _qkv(x, p, pre)
    Cm = q.shape[2]
    Cp = p[pre + '_pw0'].shape[0]
    Ca = p[pre + '_aw0'].shape[0]
    grid = (B, N // blk)
    wspec = lambda shp: pl.BlockSpec(shp, lambda b, s: (0, 0))
    return pl.pallas_call(
        functools.partial(_attn_body, K),
        grid=grid,
        in_specs=[
            pl.BlockSpec((1, blk, Cin), lambda b, s: (b, s, 0)),
            pl.BlockSpec((1, blk, 3), lambda b, s: (b, s, 0)),
            pl.BlockSpec((1, 3, N), lambda b, s: (b, 0, 0)),
            pl.BlockSpec((1, N, 3), lambda b, s: (b, 0, 0)),
            pl.BlockSpec((1, blk, Cm), lambda b, s: (b, s, 0)),
            pl.BlockSpec((1, N, Cm), lambda b, s: (b, 0, 0)),
            pl.BlockSpec((1, blk, Cm), lambda b, s: (b, s, 0)),
            wspec((3, Cp)), wspec((1, Cp)), wspec((Cp, Cm)), wspec((1, Cm)),
            wspec((Cm, Ca)), wspec((1, Ca)), wspec((Ca, Cm)), wspec((1, Cm)),
            wspec((Cm, Cin)), wspec((1, Cin)),
        ],
        out_specs=pl.BlockSpec((1, blk, Cin), lambda b, s: (b, s, 0)),
        out_shape=jax.ShapeDtypeStruct((B, N, Cin), jnp.float32),
    )(x, pos, posT, pos, q, k, v,
      jnp.transpose(p[pre + '_pw0']), p[pre + '_pb0'].reshape(1, Cp),
      jnp.transpose(p[pre + '_pw1']), p[pre + '_pb1'].reshape(1, Cm),
      jnp.transpose(p[pre + '_aw0']), p[pre + '_ab0'].reshape(1, Ca),
      jnp.transpose(p[pre + '_aw1']), p[pre + '_ab1'].reshape(1, Cm),
      jnp.transpose(p[pre + '_we']), p[pre + '_be'].reshape(1, Cin))


def _sa_all_body(xyz_ref, feats_ref, w0x_ref, w0f_ref, b0_ref, w1_ref, b1_ref,
                 out_ref):
    xyz = xyz_ref[0]
    feats = feats_ref[0]
    h = jnp.dot(xyz, w0x_ref[...], preferred_element_type=jnp.float32)
    h = h + jnp.dot(feats, w0f_ref[...], preferred_element_type=jnp.float32)
    h = jnp.maximum(h + b0_ref[...], 0.0)
    h2 = jnp.dot(h, w1_ref[...], preferred_element_type=jnp.float32) + b1_ref[...]
    out_ref[0] = jnp.max(h2, axis=0, keepdims=True)


def _sa_all(xyz, feats, w0, b0, w1, b1):
    B, S, _ = xyz.shape
    C = feats.shape[2]
    C1, C2 = w0.shape[0], w1.shape[0]
    out = pl.pallas_call(
        _sa_all_body,
        grid=(B,),
        in_specs=[
            pl.BlockSpec((1, S, 3), lambda b: (b, 0, 0)),
            pl.BlockSpec((1, S, C), lambda b: (b, 0, 0)),
            pl.BlockSpec((3, C1), lambda b: (0, 0)),
            pl.BlockSpec((C, C1), lambda b: (0, 0)),
            pl.BlockSpec((1, C1), lambda b: (0, 0)),
            pl.BlockSpec((C1, C2), lambda b: (0, 0)),
            pl.BlockSpec((1, C2), lambda b: (0, 0)),
        ],
        out_specs=pl.BlockSpec((1, 1, C2), lambda b: (b, 0, 0)),
        out_shape=jax.ShapeDtypeStruct((B, 1, C2), jnp.float32),
    )(xyz, feats, jnp.transpose(w0[:, :3]), jnp.transpose(w0[:, 3:]),
      b0.reshape(1, C1), jnp.transpose(w1), b1.reshape(1, C2))
    return out[:, 0, :]


def kernel(point_cloud, params):
    p = params
    xyz1 = point_cloud
    xyz1T = jnp.transpose(xyz1, (0, 2, 1))
    xyz1P = jnp.transpose(xyz1, (2, 0, 1))
    B, N, _ = xyz1.shape

    s1 = N // 2
    nx1P = _fps(xyz1P, s1)
    new_xyz1 = jnp.transpose(nx1P, (1, 2, 0))
    new_xyz1T = jnp.transpose(nx1P, (1, 0, 2))
    l1 = _sa_knn(new_xyz1, xyz1T, xyz1, xyz1,
                 p['sa1_w0'], p['sa1_b0'], p['sa1_w1'], p['sa1_b1'])
    l1 = _transformer(l1, new_xyz1, new_xyz1T, p, 't1')

    s2 = N // 4
    nx2P = _fps(jnp.transpose(new_xyz1, (2, 0, 1)), s2)
    new_xyz2 = jnp.transpose(nx2P, (1, 2, 0))
    new_xyz2T = jnp.transpose(nx2P, (1, 0, 2))
    l2 = _sa_knn(new_xyz2, new_xyz1T, new_xyz1, l1,
                 p['sa2_w0'], p['sa2_b0'], p['sa2_w1'], p['sa2_b1'])
    l2 = _transformer(l2, new_xyz2, new_xyz2T, p, 't2')

    return _sa_all(new_xyz2, l2, p['sa3_w0'], p['sa3_b0'], p['sa3_w1'], p['sa3_b1'])


# speedup vs baseline: 119.0133x; 1.5068x over previous
import functools

import jax
import jax.numpy as jnp
import numpy as np
from jax import lax
from jax.experimental import pallas as pl

BN_SCALE = np.float32(np.sqrt(np.float32(1.0) + np.float32(1e-5)))
_BIG = float(np.float32(3.0e38))


def _fps_body(npoint, xyz_ref, out_ref):
    x = xyz_ref[0]
    y = xyz_ref[1]
    z = xyz_ref[2]
    B, N = x.shape
    lane = lax.broadcasted_iota(jnp.int32, (B, N), 1)
    sel_lane = lax.broadcasted_iota(jnp.int32, (B, npoint), 1)

    def body(i, st):
        dist, far, ax, ay, az = st
        m = lane == far
        cx = jnp.sum(jnp.where(m, x, 0.0), axis=1, keepdims=True)
        cy = jnp.sum(jnp.where(m, y, 0.0), axis=1, keepdims=True)
        cz = jnp.sum(jnp.where(m, z, 0.0), axis=1, keepdims=True)
        keep = (sel_lane == i)
        ax = ax + jnp.where(keep, cx, 0.0)
        ay = ay + jnp.where(keep, cy, 0.0)
        az = az + jnp.where(keep, cz, 0.0)
        dx = x - cx
        dy = y - cy
        dz = z - cz
        d = dx * dx + dy * dy + dz * dz
        dist = jnp.minimum(dist, d)
        mx = jnp.max(dist, axis=1, keepdims=True)
        far = jnp.min(jnp.where(dist == mx, lane, N), axis=1, keepdims=True)
        return dist, far, ax, ay, az

    dist0 = jnp.full((B, N), 1e10, jnp.float32)
    far0 = jnp.zeros((B, 1), jnp.int32)
    acc0 = jnp.zeros((B, npoint), jnp.float32)
    _, _, ax, ay, az = lax.fori_loop(0, npoint, body, (dist0, far0, acc0, acc0, acc0))
    out_ref[0] = ax
    out_ref[1] = ay
    out_ref[2] = az


def _fps(xyz_pl, npoint):
    _, B, N = xyz_pl.shape
    return pl.pallas_call(
        functools.partial(_fps_body, npoint),
        out_shape=jax.ShapeDtypeStruct((3, B, npoint), jnp.float32),
    )(xyz_pl)


def _sa_knn_body(K, q_ref, ptsT_ref, pts_ref, feats_ref,
                 w0x_ref, w0f_ref, b0_ref, w1_ref, b1_ref, out_ref):
    q = q_ref[0]
    ptsT = ptsT_ref[0]
    pts = pts_ref[0]
    feats = feats_ref[0]
    BLK = q.shape[0]
    N = pts.shape[0]

    d = lax.dot_general(q, ptsT, (((1,), (0,)), ((), ())),
                        preferred_element_type=jnp.float32) * -2.0
    d = d + jnp.sum(q * q, axis=1, keepdims=True)
    d = d + jnp.sum(ptsT * ptsT, axis=0, keepdims=True)

    lane = lax.broadcasted_iota(jnp.int32, (BLK, N), 1)
    acc = None
    for _ in range(K):
        mn = jnp.min(d, axis=1, keepdims=True)
        sel = jnp.min(jnp.where(d == mn, lane, N), axis=1, keepdims=True)
        ohb = lane == sel
        d = jnp.where(ohb, _BIG, d)
        oh = ohb.astype(jnp.float32)
        gx = jnp.dot(oh, pts, preferred_element_type=jnp.float32)
        gf = jnp.dot(oh, feats, preferred_element_type=jnp.float32)
        rel = gx - q
        h = jnp.dot(rel, w0x_ref[...], preferred_element_type=jnp.float32)
        h = h + jnp.dot(gf, w0f_ref[...], preferred_element_type=jnp.float32)
        h = jnp.maximum(h + b0_ref[...], 0.0)
        h2 = jnp.dot(h, w1_ref[...], preferred_element_type=jnp.float32) + b1_ref[...]
        acc = h2 if acc is None else jnp.maximum(acc, h2)
    out_ref[0] = acc


def _sa_knn(new_xyz, xyzT, xyz, feats, w0, b0, w1, b1, K=16, blk=512):
    B, S, _ = new_xyz.shape
    blk = min(blk, S)
    N = xyz.shape[1]
    C = feats.shape[2]
    C1, C2 = w0.shape[0], w1.shape[0]
    w0x = jnp.transpose(w0[:, :3])
    w0f = jnp.transpose(w0[:, 3:])
    w1t = jnp.transpose(w1)
    grid = (B, S // blk)
    return pl.pallas_call(
        functools.partial(_sa_knn_body, K),
        grid=grid,
        in_specs=[
            pl.BlockSpec((1, blk, 3), lambda b, s: (b, s, 0)),
            pl.BlockSpec((1, 3, N), lambda b, s: (b, 0, 0)),
            pl.BlockSpec((1, N, 3), lambda b, s: (b, 0, 0)),
            pl.BlockSpec((1, N, C), lambda b, s: (b, 0, 0)),
            pl.BlockSpec((3, C1), lambda b, s: (0, 0)),
            pl.BlockSpec((C, C1), lambda b, s: (0, 0)),
            pl.BlockSpec((1, C1), lambda b, s: (0, 0)),
            pl.BlockSpec((C1, C2), lambda b, s: (0, 0)),
            pl.BlockSpec((1, C2), lambda b, s: (0, 0)),
        ],
        out_specs=pl.BlockSpec((1, blk, C2), lambda b, s: (b, s, 0)),
        out_shape=jax.ShapeDtypeStruct((B, S, C2), jnp.float32),
    )(new_xyz, xyzT, xyz, feats, w0x, w0f, b0.reshape(1, C1), w1t, b1.reshape(1, C2))


def _qkv_body(x_ref, ws_ref, bs_ref, wq_ref, bq_ref, wk_ref, bk_ref,
              wv_ref, bv_ref, q_ref, k_ref, v_ref):
    x = x_ref[0]
    s = jnp.dot(x, ws_ref[...], preferred_element_type=jnp.float32) + bs_ref[...]
    q_ref[0] = jnp.dot(s, wq_ref[...], preferred_element_type=jnp.float32) + bq_ref[...]
    k_ref[0] = jnp.dot(s, wk_ref[...], preferred_element_type=jnp.float32) + bk_ref[...]
    v_ref[0] = jnp.dot(s, wv_ref[...], preferred_element_type=jnp.float32) + bv_ref[...]


def _qkv(x, p, pre, blk=256):
    B, N, Cin = x.shape
    blk = min(blk, N)
    Cm = p[pre + '_ws'].shape[0]
    grid = (B, N // blk)
    wspec = lambda shp: pl.BlockSpec(shp, lambda b, s: (0, 0))
    out = pl.pallas_call(
        _qkv_body,
        grid=grid,
        in_specs=[pl.BlockSpec((1, blk, Cin), lambda b, s: (b, s, 0)),
                  wspec((Cin, Cm)), wspec((1, Cm)), wspec((Cm, Cm)), wspec((1, Cm)),
                  wspec((Cm, Cm)), wspec((1, Cm)), wspec((Cm, Cm)), wspec((1, Cm))],
        out_specs=[pl.BlockSpec((1, blk, Cm), lambda b, s: (b, s, 0))] * 3,
        out_shape=[jax.ShapeDtypeStruct((B, N, Cm), jnp.float32)] * 3,
    )(x, jnp.transpose(p[pre + '_ws']), p[pre + '_bs'].reshape(1, Cm),
      jnp.transpose(p[pre + '_wq']), p[pre + '_bq'].reshape(1, Cm),
      jnp.transpose(p[pre + '_wk']), p[pre + '_bk'].reshape(1, Cm),
      jnp.transpose(p[pre + '_wv']), p[pre + '_bv'].reshape(1, Cm))
    return out


def _attn_body(K, x_ref, pos_ref, posT_ref, posr_ref, q_ref, kall_ref, v_ref,
               pw0_ref, pb0_ref, pw1_ref, pb1_ref,
               aw0_ref, ab0_ref, aw1_ref, ab1_ref, we_ref, be_ref, out_ref):
    xid = x_ref[0]
    posq = pos_ref[0]
    posT = posT_ref[0]
    posr = posr_ref[0]
    q = q_ref[0]
    kall = kall_ref[0]
    v = v_ref[0]
    BLK = posq.shape[0]
    N = posr.shape[0]

    d = lax.dot_general(posq, posT, (((1,), (0,)), ((), ())),
                        preferred_element_type=jnp.float32) * -2.0
    d = d + jnp.sum(posq * posq, axis=1, keepdims=True)
    d = d + jnp.sum(posT * posT, axis=0, keepdims=True)

    lane = lax.broadcasted_iota(jnp.int32, (BLK, N), 1)
    logits = []
    vpes = []
    for _ in range(K):
        mn = jnp.min(d, axis=1, keepdims=True)
        sel = jnp.min(jnp.where(d == mn, lane, N), axis=1, keepdims=True)
        ohb = lane == sel
        d = jnp.where(ohb, _BIG, d)
        oh = ohb.astype(jnp.float32)
        kt = jnp.dot(oh, kall, preferred_element_type=jnp.float32)
        gpos = jnp.dot(oh, posr, preferred_element_type=jnp.float32)
        prel = posq - gpos
        pe = jnp.dot(prel, pw0_ref[...], preferred_element_type=jnp.float32) + pb0_ref[...]
        pe = jnp.maximum(pe / BN_SCALE, 0.0)
        pe = jnp.dot(pe, pw1_ref[...], preferred_element_type=jnp.float32) + pb1_ref[...]
        u = q - kt + pe
        a = jnp.dot(u, aw0_ref[...], preferred_element_type=jnp.float32) + ab0_ref[...]
        a = jnp.maximum(a / BN_SCALE, 0.0)
        a = jnp.dot(a, aw1_ref[...], preferred_element_type=jnp.float32) + ab1_ref[...]
        logits.append(a)
        vpes.append(v + pe)

    m = logits[0]
    for t in range(1, K):
        m = jnp.maximum(m, logits[t])
    num = None
    den = None
    for t in range(K):
        e = jnp.exp(logits[t] - m)
        num = e * vpes[t] if num is None else num + e * vpes[t]
        den = e if den is None else den + e
    agg = num / den
    out_ref[0] = xid + jnp.dot(agg, we_ref[...], preferred_element_type=jnp.float32) + be_ref[...]


def _transformer(x, pos, posT, p, pre, K=16, blk=512):
    B, N, Cin = x.shape
    blk = min(blk, N)
    q, k, v = _qkv(x, p, pre)
    Cm = q.shape[2]
    Cp = p[pre + '_pw0'].shape[0]
    Ca = p[pre + '_aw0'].shape[0]
    grid = (B, N // blk)
    wspec = lambda shp: pl.BlockSpec(shp, lambda b, s: (0, 0))
    return pl.pallas_call(
        functools.partial(_attn_body, K),
        grid=grid,
        in_specs=[
            pl.BlockSpec((1, blk, Cin), lambda b, s: (b, s, 0)),
            pl.BlockSpec((1, blk, 3), lambda b, s: (b, s, 0)),
            pl.BlockSpec((1, 3, N), lambda b, s: (b, 0, 0)),
            pl.BlockSpec((1, N, 3), lambda b, s: (b, 0, 0)),
            pl.BlockSpec((1, blk, Cm), lambda b, s: (b, s, 0)),
            pl.BlockSpec((1, N, Cm), lambda b, s: (b, 0, 0)),
            pl.BlockSpec((1, blk, Cm), lambda b, s: (b, s, 0)),
            wspec((3, Cp)), wspec((1, Cp)), wspec((Cp, Cm)), wspec((1, Cm)),
            wspec((Cm, Ca)), wspec((1, Ca)), wspec((Ca, Cm)), wspec((1, Cm)),
            wspec((Cm, Cin)), wspec((1, Cin)),
        ],
        out_specs=pl.BlockSpec((1, blk, Cin), lambda b, s: (b, s, 0)),
        out_shape=jax.ShapeDtypeStruct((B, N, Cin), jnp.float32),
    )(x, pos, posT, pos, q, k, v,
      jnp.transpose(p[pre + '_pw0']), p[pre + '_pb0'].reshape(1, Cp),
      jnp.transpose(p[pre + '_pw1']), p[pre + '_pb1'].reshape(1, Cm),
      jnp.transpose(p[pre + '_aw0']), p[pre + '_ab0'].reshape(1, Ca),
      jnp.transpose(p[pre + '_aw1']), p[pre + '_ab1'].reshape(1, Cm),
      jnp.transpose(p[pre + '_we']), p[pre + '_be'].reshape(1, Cin))


def _sa_all_body(xyz_ref, feats_ref, w0x_ref, w0f_ref, b0_ref, w1_ref, b1_ref,
                 out_ref):
    xyz = xyz_ref[0]
    feats = feats_ref[0]
    h = jnp.dot(xyz, w0x_ref[...], preferred_element_type=jnp.float32)
    h = h + jnp.dot(feats, w0f_ref[...], preferred_element_type=jnp.float32)
    h = jnp.maximum(h + b0_ref[...], 0.0)
    h2 = jnp.dot(h, w1_ref[...], preferred_element_type=jnp.float32) + b1_ref[...]
    out_ref[0] = jnp.max(h2, axis=0, keepdims=True)


def _sa_all(xyz, feats, w0, b0, w1, b1):
    B, S, _ = xyz.shape
    C = feats.shape[2]
    C1, C2 = w0.shape[0], w1.shape[0]
    out = pl.pallas_call(
        _sa_all_body,
        grid=(B,),
        in_specs=[
            pl.BlockSpec((1, S, 3), lambda b: (b, 0, 0)),
            pl.BlockSpec((1, S, C), lambda b: (b, 0, 0)),
            pl.BlockSpec((3, C1), lambda b: (0, 0)),
            pl.BlockSpec((C, C1), lambda b: (0, 0)),
            pl.BlockSpec((1, C1), lambda b: (0, 0)),
            pl.BlockSpec((C1, C2), lambda b: (0, 0)),
            pl.BlockSpec((1, C2), lambda b: (0, 0)),
        ],
        out_specs=pl.BlockSpec((1, 1, C2), lambda b: (b, 0, 0)),
        out_shape=jax.ShapeDtypeStruct((B, 1, C2), jnp.float32),
    )(xyz, feats, jnp.transpose(w0[:, :3]), jnp.transpose(w0[:, 3:]),
      b0.reshape(1, C1), jnp.transpose(w1), b1.reshape(1, C2))
    return out[:, 0, :]


def kernel(point_cloud, params):
    p = params
    xyz1 = point_cloud
    xyz1T = jnp.transpose(xyz1, (0, 2, 1))
    xyz1P = jnp.transpose(xyz1, (2, 0, 1))
    B, N, _ = xyz1.shape

    s1 = N // 2
    nx1P = _fps(xyz1P, s1)
    new_xyz1 = jnp.transpose(nx1P, (1, 2, 0))
    new_xyz1T = jnp.transpose(nx1P, (1, 0, 2))
    l1 = _sa_knn(new_xyz1, xyz1T, xyz1, xyz1,
                 p['sa1_w0'], p['sa1_b0'], p['sa1_w1'], p['sa1_b1'])
    l1 = _transformer(l1, new_xyz1, new_xyz1T, p, 't1')

    s2 = N // 4
    nx2P = _fps(jnp.transpose(new_xyz1, (2, 0, 1)), s2)
    new_xyz2 = jnp.transpose(nx2P, (1, 2, 0))
    new_xyz2T = jnp.transpose(nx2P, (1, 0, 2))
    l2 = _sa_knn(new_xyz2, new_xyz1T, new_xyz1, l1,
                 p['sa2_w0'], p['sa2_b0'], p['sa2_w1'], p['sa2_b1'])
    l2 = _transformer(l2, new_xyz2, new_xyz2T, p, 't2')

    return _sa_all(new_xyz2, l2, p['sa3_w0'], p['sa3_b0'], p['sa3_w1'], p['sa3_b1'])


# speedup vs baseline: 130.3447x; 1.0952x over previous
import functools

import jax
import jax.numpy as jnp
import numpy as np
from jax import lax
from jax.experimental import pallas as pl

BN_SCALE = np.float32(np.sqrt(np.float32(1.0) + np.float32(1e-5)))
_BIG = float(np.float32(3.0e38))


def _fps_body(npoint, xyz_ref, out_ref):
    x = xyz_ref[0]
    y = xyz_ref[1]
    z = xyz_ref[2]
    B, N = x.shape
    lane = lax.broadcasted_iota(jnp.int32, (B, N), 1)
    sel_lane = lax.broadcasted_iota(jnp.int32, (B, npoint), 1)

    def body(i, st):
        dist, far, ax, ay, az = st
        m = lane == far
        cx = jnp.sum(jnp.where(m, x, 0.0), axis=1, keepdims=True)
        cy = jnp.sum(jnp.where(m, y, 0.0), axis=1, keepdims=True)
        cz = jnp.sum(jnp.where(m, z, 0.0), axis=1, keepdims=True)
        keep = (sel_lane == i)
        ax = ax + jnp.where(keep, cx, 0.0)
        ay = ay + jnp.where(keep, cy, 0.0)
        az = az + jnp.where(keep, cz, 0.0)
        dx = x - cx
        dy = y - cy
        dz = z - cz
        d = dx * dx + dy * dy + dz * dz
        dist = jnp.minimum(dist, d)
        far = jnp.argmax(dist, axis=1).astype(jnp.int32).reshape(B, 1)
        return dist, far, ax, ay, az

    dist0 = jnp.full((B, N), 1e10, jnp.float32)
    far0 = jnp.zeros((B, 1), jnp.int32)
    acc0 = jnp.zeros((B, npoint), jnp.float32)
    _, _, ax, ay, az = lax.fori_loop(0, npoint, body, (dist0, far0, acc0, acc0, acc0))
    out_ref[0] = ax
    out_ref[1] = ay
    out_ref[2] = az


def _fps(xyz_pl, npoint):
    _, B, N = xyz_pl.shape
    return pl.pallas_call(
        functools.partial(_fps_body, npoint),
        out_shape=jax.ShapeDtypeStruct((3, B, npoint), jnp.float32),
    )(xyz_pl)


def _sa_knn_body(K, q_ref, ptsT_ref, pts_ref, feats_ref,
                 w0x_ref, w0f_ref, b0_ref, w1_ref, b1_ref, out_ref):
    q = q_ref[0]
    ptsT = ptsT_ref[0]
    pts = pts_ref[0]
    feats = feats_ref[0]
    BLK = q.shape[0]
    N = pts.shape[0]

    d = lax.dot_general(q, ptsT, (((1,), (0,)), ((), ())),
                        preferred_element_type=jnp.float32) * -2.0
    d = d + jnp.sum(q * q, axis=1, keepdims=True)
    d = d + jnp.sum(ptsT * ptsT, axis=0, keepdims=True)

    lane = lax.broadcasted_iota(jnp.int32, (BLK, N), 1)
    acc = None
    for _ in range(K):
        sel = jnp.argmin(d, axis=1).astype(jnp.int32).reshape(BLK, 1)
        ohb = lane == sel
        d = jnp.where(ohb, _BIG, d)
        oh = ohb.astype(jnp.float32)
        gx = jnp.dot(oh, pts, preferred_element_type=jnp.float32)
        gf = jnp.dot(oh, feats, preferred_element_type=jnp.float32)
        rel = gx - q
        h = jnp.dot(rel, w0x_ref[...], preferred_element_type=jnp.float32)
        h = h + jnp.dot(gf, w0f_ref[...], preferred_element_type=jnp.float32)
        h = jnp.maximum(h + b0_ref[...], 0.0)
        h2 = jnp.dot(h, w1_ref[...], preferred_element_type=jnp.float32) + b1_ref[...]
        acc = h2 if acc is None else jnp.maximum(acc, h2)
    out_ref[0] = acc


def _sa_knn(new_xyz, xyzT, xyz, feats, w0, b0, w1, b1, K=16, blk=512):
    B, S, _ = new_xyz.shape
    blk = min(blk, S)
    N = xyz.shape[1]
    C = feats.shape[2]
    C1, C2 = w0.shape[0], w1.shape[0]
    w0x = jnp.transpose(w0[:, :3])
    w0f = jnp.transpose(w0[:, 3:])
    w1t = jnp.transpose(w1)
    grid = (B, S // blk)
    return pl.pallas_call(
        functools.partial(_sa_knn_body, K),
        grid=grid,
        in_specs=[
            pl.BlockSpec((1, blk, 3), lambda b, s: (b, s, 0)),
            pl.BlockSpec((1, 3, N), lambda b, s: (b, 0, 0)),
            pl.BlockSpec((1, N, 3), lambda b, s: (b, 0, 0)),
            pl.BlockSpec((1, N, C), lambda b, s: (b, 0, 0)),
            pl.BlockSpec((3, C1), lambda b, s: (0, 0)),
            pl.BlockSpec((C, C1), lambda b, s: (0, 0)),
            pl.BlockSpec((1, C1), lambda b, s: (0, 0)),
            pl.BlockSpec((C1, C2), lambda b, s: (0, 0)),
            pl.BlockSpec((1, C2), lambda b, s: (0, 0)),
        ],
        out_specs=pl.BlockSpec((1, blk, C2), lambda b, s: (b, s, 0)),
        out_shape=jax.ShapeDtypeStruct((B, S, C2), jnp.float32),
    )(new_xyz, xyzT, xyz, feats, w0x, w0f, b0.reshape(1, C1), w1t, b1.reshape(1, C2))


def _qkv_body(x_ref, ws_ref, bs_ref, wq_ref, bq_ref, wk_ref, bk_ref,
              wv_ref, bv_ref, q_ref, k_ref, v_ref):
    x = x_ref[0]
    s = jnp.dot(x, ws_ref[...], preferred_element_type=jnp.float32) + bs_ref[...]
    q_ref[0] = jnp.dot(s, wq_ref[...], preferred_element_type=jnp.float32) + bq_ref[...]
    k_ref[0] = jnp.dot(s, wk_ref[...], preferred_element_type=jnp.float32) + bk_ref[...]
    v_ref[0] = jnp.dot(s, wv_ref[...], preferred_element_type=jnp.float32) + bv_ref[...]


def _qkv(x, p, pre, blk=256):
    B, N, Cin = x.shape
    blk = min(blk, N)
    Cm = p[pre + '_ws'].shape[0]
    grid = (B, N // blk)
    wspec = lambda shp: pl.BlockSpec(shp, lambda b, s: (0, 0))
    out = pl.pallas_call(
        _qkv_body,
        grid=grid,
        in_specs=[pl.BlockSpec((1, blk, Cin), lambda b, s: (b, s, 0)),
                  wspec((Cin, Cm)), wspec((1, Cm)), wspec((Cm, Cm)), wspec((1, Cm)),
                  wspec((Cm, Cm)), wspec((1, Cm)), wspec((Cm, Cm)), wspec((1, Cm))],
        out_specs=[pl.BlockSpec((1, blk, Cm), lambda b, s: (b, s, 0))] * 3,
        out_shape=[jax.ShapeDtypeStruct((B, N, Cm), jnp.float32)] * 3,
    )(x, jnp.transpose(p[pre + '_ws']), p[pre + '_bs'].reshape(1, Cm),
      jnp.transpose(p[pre + '_wq']), p[pre + '_bq'].reshape(1, Cm),
      jnp.transpose(p[pre + '_wk']), p[pre + '_bk'].reshape(1, Cm),
      jnp.transpose(p[pre + '_wv']), p[pre + '_bv'].reshape(1, Cm))
    return out


def _attn_body(K, x_ref, pos_ref, posT_ref, posr_ref, q_ref, kall_ref, v_ref,
               pw0_ref, pb0_ref, pw1_ref, pb1_ref,
               aw0_ref, ab0_ref, aw1_ref, ab1_ref, we_ref, be_ref, out_ref):
    xid = x_ref[0]
    posq = pos_ref[0]
    posT = posT_ref[0]
    posr = posr_ref[0]
    q = q_ref[0]
    kall = kall_ref[0]
    v = v_ref[0]
    BLK = posq.shape[0]
    N = posr.shape[0]

    d = lax.dot_general(posq, posT, (((1,), (0,)), ((), ())),
                        preferred_element_type=jnp.float32) * -2.0
    d = d + jnp.sum(posq * posq, axis=1, keepdims=True)
    d = d + jnp.sum(posT * posT, axis=0, keepdims=True)

    lane = lax.broadcasted_iota(jnp.int32, (BLK, N), 1)
    logits = []
    vpes = []
    for _ in range(K):
        sel = jnp.argmin(d, axis=1).astype(jnp.int32).reshape(BLK, 1)
        ohb = lane == sel
        d = jnp.where(ohb, _BIG, d)
        oh = ohb.astype(jnp.float32)
        kt = jnp.dot(oh, kall, preferred_element_type=jnp.float32)
        gpos = jnp.dot(oh, posr, preferred_element_type=jnp.float32)
        prel = posq - gpos
        pe = jnp.dot(prel, pw0_ref[...], preferred_element_type=jnp.float32) + pb0_ref[...]
        pe = jnp.maximum(pe / BN_SCALE, 0.0)
        pe = jnp.dot(pe, pw1_ref[...], preferred_element_type=jnp.float32) + pb1_ref[...]
        u = q - kt + pe
        a = jnp.dot(u, aw0_ref[...], preferred_element_type=jnp.float32) + ab0_ref[...]
        a = jnp.maximum(a / BN_SCALE, 0.0)
        a = jnp.dot(a, aw1_ref[...], preferred_element_type=jnp.float32) + ab1_ref[...]
        logits.append(a)
        vpes.append(v + pe)

    m = logits[0]
    for t in range(1, K):
        m = jnp.maximum(m, logits[t])
    num = None
    den = None
    for t in range(K):
        e = jnp.exp(logits[t] - m)
        num = e * vpes[t] if num is None else num + e * vpes[t]
        den = e if den is None else den + e
    agg = num / den
    out_ref[0] = xid + jnp.dot(agg, we_ref[...], preferred_element_type=jnp.float32) + be_ref[...]


def _transformer(x, pos, posT, p, pre, K=16, blk=512):
    B, N, Cin = x.shape
    blk = min(blk, N)
    q, k, v = _qkv(x, p, pre)
    Cm = q.shape[2]
    Cp = p[pre + '_pw0'].shape[0]
    Ca = p[pre + '_aw0'].shape[0]
    grid = (B, N // blk)
    wspec = lambda shp: pl.BlockSpec(shp, lambda b, s: (0, 0))
    return pl.pallas_call(
        functools.partial(_attn_body, K),
        grid=grid,
        in_specs=[
            pl.BlockSpec((1, blk, Cin), lambda b, s: (b, s, 0)),
            pl.BlockSpec((1, blk, 3), lambda b, s: (b, s, 0)),
            pl.BlockSpec((1, 3, N), lambda b, s: (b, 0, 0)),
            pl.BlockSpec((1, N, 3), lambda b, s: (b, 0, 0)),
            pl.BlockSpec((1, blk, Cm), lambda b, s: (b, s, 0)),
            pl.BlockSpec((1, N, Cm), lambda b, s: (b, 0, 0)),
            pl.BlockSpec((1, blk, Cm), lambda b, s: (b, s, 0)),
            wspec((3, Cp)), wspec((1, Cp)), wspec((Cp, Cm)), wspec((1, Cm)),
            wspec((Cm, Ca)), wspec((1, Ca)), wspec((Ca, Cm)), wspec((1, Cm)),
            wspec((Cm, Cin)), wspec((1, Cin)),
        ],
        out_specs=pl.BlockSpec((1, blk, Cin), lambda b, s: (b, s, 0)),
        out_shape=jax.ShapeDtypeStruct((B, N, Cin), jnp.float32),
    )(x, pos, posT, pos, q, k, v,
      jnp.transpose(p[pre + '_pw0']), p[pre + '_pb0'].reshape(1, Cp),
      jnp.transpose(p[pre + '_pw1']), p[pre + '_pb1'].reshape(1, Cm),
      jnp.transpose(p[pre + '_aw0']), p[pre + '_ab0'].reshape(1, Ca),
      jnp.transpose(p[pre + '_aw1']), p[pre + '_ab1'].reshape(1, Cm),
      jnp.transpose(p[pre + '_we']), p[pre + '_be'].reshape(1, Cin))


def _sa_all_body(xyz_ref, feats_ref, w0x_ref, w0f_ref, b0_ref, w1_ref, b1_ref,
                 out_ref):
    xyz = xyz_ref[0]
    feats = feats_ref[0]
    h = jnp.dot(xyz, w0x_ref[...], preferred_element_type=jnp.float32)
    h = h + jnp.dot(feats, w0f_ref[...], preferred_element_type=jnp.float32)
    h = jnp.maximum(h + b0_ref[...], 0.0)
    h2 = jnp.dot(h, w1_ref[...], preferred_element_type=jnp.float32) + b1_ref[...]
    out_ref[0] = jnp.max(h2, axis=0, keepdims=True)


def _sa_all(xyz, feats, w0, b0, w1, b1):
    B, S, _ = xyz.shape
    C = feats.shape[2]
    C1, C2 = w0.shape[0], w1.shape[0]
    out = pl.pallas_call(
        _sa_all_body,
        grid=(B,),
        in_specs=[
            pl.BlockSpec((1, S, 3), lambda b: (b, 0, 0)),
            pl.BlockSpec((1, S, C), lambda b: (b, 0, 0)),
            pl.BlockSpec((3, C1), lambda b: (0, 0)),
            pl.BlockSpec((C, C1), lambda b: (0, 0)),
            pl.BlockSpec((1, C1), lambda b: (0, 0)),
            pl.BlockSpec((C1, C2), lambda b: (0, 0)),
            pl.BlockSpec((1, C2), lambda b: (0, 0)),
        ],
        out_specs=pl.BlockSpec((1, 1, C2), lambda b: (b, 0, 0)),
        out_shape=jax.ShapeDtypeStruct((B, 1, C2), jnp.float32),
    )(xyz, feats, jnp.transpose(w0[:, :3]), jnp.transpose(w0[:, 3:]),
      b0.reshape(1, C1), jnp.transpose(w1), b1.reshape(1, C2))
    return out[:, 0, :]


def kernel(point_cloud, params):
    p = params
    xyz1 = point_cloud
    xyz1T = jnp.transpose(xyz1, (0, 2, 1))
    xyz1P = jnp.transpose(xyz1, (2, 0, 1))
    B, N, _ = xyz1.shape

    s1 = N // 2
    nx1P = _fps(xyz1P, s1)
    new_xyz1 = jnp.transpose(nx1P, (1, 2, 0))
    new_xyz1T = jnp.transpose(nx1P, (1, 0, 2))
    l1 = _sa_knn(new_xyz1, xyz1T, xyz1, xyz1,
                 p['sa1_w0'], p['sa1_b0'], p['sa1_w1'], p['sa1_b1'])
    l1 = _transformer(l1, new_xyz1, new_xyz1T, p, 't1')

    s2 = N // 4
    nx2P = _fps(jnp.transpose(new_xyz1, (2, 0, 1)), s2)
    new_xyz2 = jnp.transpose(nx2P, (1, 2, 0))
    new_xyz2T = jnp.transpose(nx2P, (1, 0, 2))
    l2 = _sa_knn(new_xyz2, new_xyz1T, new_xyz1, l1,
                 p['sa2_w0'], p['sa2_b0'], p['sa2_w1'], p['sa2_b1'])
    l2 = _transformer(l2, new_xyz2, new_xyz2T, p, 't2')

    return _sa_all(new_xyz2, l2, p['sa3_w0'], p['sa3_b0'], p['sa3_w1'], p['sa3_b1'])
